# Initial kernel scaffold; baseline (speedup 1.0000x reference)
import jax
import jax.numpy as jnp
from jax.experimental import pallas as pl


def kernel(scores, deltas, anchors):
    raise NotImplementedError("write your pallas kernel here")


# speedup vs baseline: 31.6737x; 31.6737x over previous
import jax
import jax.numpy as jnp
from jax import lax
from jax.experimental import pallas as pl
from jax.experimental.pallas import tpu as pltpu

_F32 = jnp.float32
_PROPOSALS = 1000
_PRE = 6000
_TAU = 0.7
_K = 256
_S = 6144
_NB = _S // _K
_P = 1024
_STD = (0.1, 0.1, 0.2, 0.2)
_INTERPRET = False

_DN = (((1,), (0,)), ((), ()))


def _colT(v):
    return jnp.swapaxes(v, 0, 1)


def _suppress_ind(cols, rows):
    cy1, cx1, cy2, cx2, car = cols
    ry1, rx1, ry2, rx2, rar = rows
    yy1 = jnp.maximum(ry1, cy1)
    xx1 = jnp.maximum(rx1, cx1)
    yy2 = jnp.minimum(ry2, cy2)
    xx2 = jnp.minimum(rx2, cx2)
    inter = jnp.maximum(yy2 - yy1, 0.0) * jnp.maximum(xx2 - xx1, 0.0)
    iou = inter / (rar + car - inter + 1e-8)
    return iou > _TAU


def _nms_body(at_ref, dt_ref, out_ref, crd, keeps):
    a = at_ref[0]
    d = dt_ref[0]
    a0, a1, a2, a3 = a[0:1], a[1:2], a[2:3], a[3:4]
    d0, d1, d2, d3 = d[0:1], d[1:2], d[2:3], d[3:4]
    h = a2 - a0
    w = a3 - a1
    cy = (a0 + 0.5 * h) + d0 * h
    cx = (a1 + 0.5 * w) + d1 * w
    he = h * jnp.exp(d2)
    we = w * jnp.exp(d3)
    y1 = cy - 0.5 * he
    x1 = cx - 0.5 * we
    y2 = y1 + he
    x2 = x1 + we
    y1 = jnp.clip(y1, 0.0, 1.0)
    x1 = jnp.clip(x1, 0.0, 1.0)
    y2 = jnp.clip(y2, 0.0, 1.0)
    x2 = jnp.clip(x2, 0.0, 1.0)
    ar = (y2 - y1) * (x2 - x1)
    crd[0:1, :] = y1
    crd[1:2, :] = x1
    crd[2:3, :] = y2
    crd[3:4, :] = x2
    crd[4:5, :] = ar

    out_ref[0] = jnp.zeros((4, _P), _F32)

    io_s = lax.broadcasted_iota(jnp.int32, (_K, _K), 0)
    io_l = lax.broadcasted_iota(jnp.int32, (_K, _K), 1)
    lt = (io_l < io_s).astype(_F32)
    p_row = lax.broadcasted_iota(jnp.int32, (1, _P), 1)
    blk_iota = lax.broadcasted_iota(jnp.int32, (_K, 1), 0)

    def get_rows(base):
        sl = pl.ds(base, _K)
        return (crd[0:1, sl], crd[1:2, sl], crd[2:3, sl], crd[3:4, sl],
                crd[4:5, sl])

    def process(b, kc):
        base = pl.multiple_of(b * _K, _K)
        rows_b = get_rows(base)
        cols_b = tuple(_colT(r) for r in rows_b)

        def cross(c, acc):
            cb = pl.multiple_of(c * _K, _K)
            ind = _suppress_ind(cols_b, get_rows(cb)).astype(_F32)
            kcol = keeps[pl.ds(cb, _K), :]
            return acc + lax.dot_general(ind, kcol, _DN,
                                         preferred_element_type=_F32)

        sup = lax.fori_loop(0, b, cross, jnp.zeros((_K, 1), _F32))
        cand = ((blk_iota + base) < _PRE).astype(_F32)
        base_keep = cand * (sup < 0.5).astype(_F32)

        mb = _suppress_ind(cols_b, rows_b).astype(_F32) * lt

        def fix_cond(st):
            return st[1]

        def fix_body(st):
            k = st[0]
            s = lax.dot_general(mb, k, _DN, preferred_element_type=_F32)
            nk = base_keep * (s < 0.5).astype(_F32)
            return nk, jnp.any(nk != k)

        keep_b, _ = lax.while_loop(fix_cond, fix_body, (base_keep, True))
        keeps[pl.ds(base, _K), :] = keep_b

        rank = lax.dot_general(lt, keep_b, _DN, preferred_element_type=_F32)
        pos = (kc + rank).astype(jnp.int32)
        sel = ((pos == p_row) & (keep_b > 0.5)).astype(_F32)
        boxr = jnp.concatenate(rows_b[:4], axis=0)
        out_ref[0] += lax.dot_general(boxr, sel, _DN,
                                      preferred_element_type=_F32)
        return kc + jnp.sum(keep_b)

    def blk(b, kc):
        return lax.cond(kc < float(_PROPOSALS), process,
                        lambda b_, k_: k_, b, kc)

    lax.fori_loop(0, _NB, blk, 0.0)


def kernel(scores, deltas, anchors):
    B, N, _ = scores.shape
    fg = scores[:, :, 1]
    ds = deltas * jnp.asarray(_STD, _F32).reshape(1, 1, 4)
    top_s, top_i = lax.top_k(fg, _PRE)
    del top_s
    d = jnp.take_along_axis(ds, top_i[:, :, None], axis=1)
    a = jnp.take_along_axis(anchors, top_i[:, :, None], axis=1)
    pad = ((0, 0), (0, 0), (0, _S - _PRE))
    at = jnp.pad(jnp.transpose(a, (0, 2, 1)), pad)
    dt = jnp.pad(jnp.transpose(d, (0, 2, 1)), pad)

    out = pl.pallas_call(
        _nms_body,
        grid=(B,),
        in_specs=[
            pl.BlockSpec((1, 4, _S), lambda b: (b, 0, 0)),
            pl.BlockSpec((1, 4, _S), lambda b: (b, 0, 0)),
        ],
        out_specs=pl.BlockSpec((1, 4, _P), lambda b: (b, 0, 0)),
        out_shape=jax.ShapeDtypeStruct((B, 4, _P), _F32),
        scratch_shapes=[
            pltpu.VMEM((8, _S), _F32),
            pltpu.VMEM((_S, 1), _F32),
        ],
        interpret=_INTERPRET,
    )(at, dt)
    return jnp.transpose(out, (0, 2, 1))[:, :_PROPOSALS, :]
